# baseline (speedup 1.0000x reference)
import functools

import jax
import jax.numpy as jnp
from jax import lax
from jax.experimental import pallas as pl
from jax.experimental.pallas import tpu as pltpu
from jax.experimental.pallas import tpu_sc as plsc

N = 10000
E = 320000
D_IN = 128
DH = 64

NC = 2
NS = 16
NW = NC * NS
CHUNK = 128
NCH = 80
E_PAD = NW * NCH * CHUNK
NP = 10112
RPT = NP // NS
MB = 8

_MESH = plsc.VectorSubcoreMesh(core_axis_name="c", subcore_axis_name="s")


def _deg_body(dst_hbm, ones_hbm, z16_hbm, out_hbm, didx_v, ones_v, acc_sh, sem):
    c = lax.axis_index("c")
    s = lax.axis_index("s")
    wid = c * NS + s
    base = s * RPT
    pltpu.sync_copy(dst_hbm.at[wid], didx_v)
    pltpu.sync_copy(ones_hbm, ones_v)
    pltpu.sync_copy(z16_hbm.at[pl.ds(base, RPT)], acc_sh.at[pl.ds(base, RPT)])
    plsc.subcore_barrier()

    def body(j, carry):
        pltpu.sync_copy(ones_v, acc_sh.at[didx_v.at[j]], add=True)
        return carry

    lax.fori_loop(0, NCH, body, 0)
    plsc.subcore_barrier()
    pltpu.sync_copy(acc_sh.at[pl.ds(base, RPT)], out_hbm.at[c, pl.ds(base, RPT)])


@functools.partial(
    pl.kernel,
    out_type=jax.ShapeDtypeStruct((NC, NP, 16), jnp.float32),
    mesh=_MESH,
    compiler_params=pltpu.CompilerParams(use_tc_tiling_on_sc=False),
    scratch_types=[
        pltpu.VMEM((NCH, CHUNK), jnp.int32),
        pltpu.VMEM((CHUNK, 16), jnp.float32),
        pltpu.VMEM_SHARED((NP, 16), jnp.float32),
        pltpu.SemaphoreType.DMA,
    ],
)
def _deg_kernel(dst_hbm, ones_hbm, z16_hbm, out_hbm, didx_v, ones_v, acc_sh, sem):
    _deg_body(dst_hbm, ones_hbm, z16_hbm, out_hbm, didx_v, ones_v, acc_sh, sem)


def _mp_body(hs_hbm, src_hbm, dst_hbm, z64_hbm, out_hbm,
             sidx_v, didx_v, gbuf, acc_sh, sem):
    c = lax.axis_index("c")
    s = lax.axis_index("s")
    wid = c * NS + s
    base = s * RPT
    pltpu.sync_copy(src_hbm.at[wid], sidx_v)
    pltpu.sync_copy(dst_hbm.at[wid], didx_v)
    pltpu.sync_copy(z64_hbm.at[pl.ds(base, RPT)], acc_sh.at[pl.ds(base, RPT)])
    plsc.subcore_barrier()

    def body(j, carry):
        blk = pl.ds(j * MB * CHUNK, MB * CHUNK)
        pltpu.async_copy(hs_hbm.at[sidx_v.at[blk]], gbuf, sem).wait()
        pltpu.sync_copy(gbuf, acc_sh.at[didx_v.at[blk]], add=True)
        return carry

    lax.fori_loop(0, NCH // MB, body, 0)
    plsc.subcore_barrier()
    pltpu.sync_copy(acc_sh.at[pl.ds(base, RPT)], out_hbm.at[c, pl.ds(base, RPT)])


@functools.partial(
    pl.kernel,
    out_type=jax.ShapeDtypeStruct((NC, NP, DH), jnp.float32),
    mesh=_MESH,
    compiler_params=pltpu.CompilerParams(use_tc_tiling_on_sc=False),
    scratch_types=[
        pltpu.VMEM((NCH * CHUNK,), jnp.int32),
        pltpu.VMEM((NCH * CHUNK,), jnp.int32),
        pltpu.VMEM((MB * CHUNK, DH), jnp.float32),
        pltpu.VMEM_SHARED((NP, DH), jnp.float32),
        pltpu.SemaphoreType.DMA,
    ],
)
def _mp_kernel(hs_hbm, src_hbm, dst_hbm, z64_hbm, out_hbm,
               sidx_v, didx_v, gbuf, acc_sh, sem):
    _mp_body(hs_hbm, src_hbm, dst_hbm, z64_hbm, out_hbm,
             sidx_v, didx_v, gbuf, acc_sh, sem)


def _tc_a_body(x_ref, w1_ref, dp_ref, hs_ref, dis_ref):
    deg = dp_ref[0, 0:N, 0:1] + dp_ref[1, 0:N, 0:1] + 1.0
    dis = lax.rsqrt(deg)
    dis_ref[...] = dis
    h = jnp.dot(x_ref[...], w1_ref[...], preferred_element_type=jnp.float32)
    hs_ref[0:N, :] = (dis * h).astype(jnp.float32)
    hs_ref[N:NP, :] = jnp.zeros((NP - N, DH), jnp.float32)


def _tc_mid_body(p_ref, hs_ref, dis_ref, b_ref, g_ref, be_ref, w2_ref, out_ref,
                 *, eps=1e-5):
    dis = dis_ref[...]
    acc = (p_ref[0, 0:N, :]
           + p_ref[1, 0:N, :].astype(jnp.float32)
           + hs_ref[0:N, :].astype(jnp.float32))
    z = dis * acc + b_ref[...]
    m = jnp.mean(z, axis=0, keepdims=True)
    v = jnp.mean((z - m) ** 2, axis=0, keepdims=True)
    zn = g_ref[...] * (z - m) * lax.rsqrt(v + eps) + be_ref[...]
    h = jnp.maximum(zn, 0.0)
    h2 = jnp.dot(h, w2_ref[...], preferred_element_type=jnp.float32)
    out_ref[0:N, :] = (dis * h2).astype(jnp.float32)
    out_ref[N:NP, :] = jnp.zeros((NP - N, DH), jnp.float32)


def _tc_out_body(p_ref, hs_ref, dis_ref, b_ref, g_ref, be_ref, out_ref,
                 *, eps=1e-5):
    dis = dis_ref[...]
    acc = (p_ref[0, 0:N, :]
           + p_ref[1, 0:N, :].astype(jnp.float32)
           + hs_ref[0:N, :].astype(jnp.float32))
    z = dis * acc + b_ref[...]
    m = jnp.mean(z, axis=0, keepdims=True)
    v = jnp.mean((z - m) ** 2, axis=0, keepdims=True)
    zn = g_ref[...] * (z - m) * lax.rsqrt(v + eps) + be_ref[...]
    out_ref[...] = jnp.maximum(zn, 0.0)


def kernel(x, edge_index, W1, b1, gamma1, beta1, W2, b2, gamma2, beta2):
    pad = jnp.full((E_PAD - E,), N, dtype=jnp.int32)
    src = jnp.concatenate([edge_index[0], pad]).reshape(NW, NCH * CHUNK)
    dst = jnp.concatenate([edge_index[1], pad]).reshape(NW, NCH * CHUNK)
    dst_deg = dst.reshape(NW, NCH, CHUNK)
    ones16 = jnp.ones((CHUNK, 16), jnp.float32)
    z16 = jnp.zeros((NP, 16), jnp.float32)
    z64 = jnp.zeros((NP, DH), jnp.float32)

    degpart = _deg_kernel(dst_deg, ones16, z16)

    hs1, dis = pl.pallas_call(
        _tc_a_body,
        out_shape=(jax.ShapeDtypeStruct((NP, DH), jnp.float32),
                   jax.ShapeDtypeStruct((N, 1), jnp.float32)),
    )(x, W1, degpart)

    p1 = _mp_kernel(hs1, src, dst, z64)

    hs2 = pl.pallas_call(
        _tc_mid_body,
        out_shape=jax.ShapeDtypeStruct((NP, DH), jnp.float32),
    )(p1, hs1, dis, b1, gamma1, beta1, W2)

    p2 = _mp_kernel(hs2, src, dst, z64)

    out = pl.pallas_call(
        _tc_out_body,
        out_shape=jax.ShapeDtypeStruct((N, DH), jnp.float32),
    )(p2, hs2, dis, b2, gamma2, beta2)

    return out


# speedup vs baseline: 1.7153x; 1.7153x over previous
import functools

import jax
import jax.numpy as jnp
from jax import lax
from jax.experimental import pallas as pl
from jax.experimental.pallas import tpu as pltpu
from jax.experimental.pallas import tpu_sc as plsc

N = 10000
E = 320000
D_IN = 128
DH = 64

NC = 2
NS = 16
NW = NC * NS
CHUNK = 128
NCH = 80
E_PAD = NW * NCH * CHUNK
NP = 10112
RPT = NP // NS
MB = 1

_MESH = plsc.VectorSubcoreMesh(core_axis_name="c", subcore_axis_name="s")


def _deg_body(dst_hbm, ones_hbm, z16_hbm, out_hbm, didx_v, ones_v, acc_sh, sem):
    c = lax.axis_index("c")
    s = lax.axis_index("s")
    wid = c * NS + s
    base = s * RPT
    pltpu.sync_copy(dst_hbm.at[wid], didx_v)
    pltpu.sync_copy(ones_hbm, ones_v)
    pltpu.sync_copy(z16_hbm.at[pl.ds(base, RPT)], acc_sh.at[pl.ds(base, RPT)])
    plsc.subcore_barrier()

    def body(j, carry):
        pltpu.sync_copy(ones_v, acc_sh.at[didx_v.at[j]], add=True)
        return carry

    lax.fori_loop(0, NCH, body, 0)
    plsc.subcore_barrier()
    pltpu.sync_copy(acc_sh.at[pl.ds(base, RPT)], out_hbm.at[c, pl.ds(base, RPT)])


@functools.partial(
    pl.kernel,
    out_type=jax.ShapeDtypeStruct((NC, NP, 16), jnp.float32),
    mesh=_MESH,
    compiler_params=pltpu.CompilerParams(use_tc_tiling_on_sc=False),
    scratch_types=[
        pltpu.VMEM((NCH, CHUNK), jnp.int32),
        pltpu.VMEM((CHUNK, 16), jnp.float32),
        pltpu.VMEM_SHARED((NP, 16), jnp.float32),
        pltpu.SemaphoreType.DMA,
    ],
)
def _deg_kernel(dst_hbm, ones_hbm, z16_hbm, out_hbm, didx_v, ones_v, acc_sh, sem):
    _deg_body(dst_hbm, ones_hbm, z16_hbm, out_hbm, didx_v, ones_v, acc_sh, sem)


def _mp_body(hs_hbm, src_hbm, dst_hbm, z64_hbm, out_hbm,
             sidx_v, didx_v, gbuf, hs_sh, acc_sh, sem):
    c = lax.axis_index("c")
    s = lax.axis_index("s")
    wid = c * NS + s
    base = s * RPT
    pltpu.sync_copy(src_hbm.at[wid], sidx_v)
    pltpu.sync_copy(dst_hbm.at[wid], didx_v)
    pltpu.sync_copy(hs_hbm.at[pl.ds(base, RPT)], hs_sh.at[pl.ds(base, RPT)])
    pltpu.sync_copy(z64_hbm.at[pl.ds(base, RPT)], acc_sh.at[pl.ds(base, RPT)])
    plsc.subcore_barrier()

    def body(j, carry):
        blk = pl.ds(j * MB * CHUNK, MB * CHUNK)
        pltpu.async_copy(hs_sh.at[sidx_v.at[blk]], gbuf, sem).wait()
        pltpu.sync_copy(gbuf, acc_sh.at[didx_v.at[blk]], add=True)
        return carry

    lax.fori_loop(0, NCH // MB, body, 0)
    plsc.subcore_barrier()
    pltpu.sync_copy(acc_sh.at[pl.ds(base, RPT)], out_hbm.at[c, pl.ds(base, RPT)])


@functools.partial(
    pl.kernel,
    out_type=jax.ShapeDtypeStruct((NC, NP, DH), jnp.float32),
    mesh=_MESH,
    compiler_params=pltpu.CompilerParams(use_tc_tiling_on_sc=False),
    scratch_types=[
        pltpu.VMEM((NCH * CHUNK,), jnp.int32),
        pltpu.VMEM((NCH * CHUNK,), jnp.int32),
        pltpu.VMEM((MB * CHUNK, DH), jnp.float32),
        pltpu.VMEM_SHARED((NP, DH), jnp.float32),
        pltpu.VMEM_SHARED((NP, DH), jnp.float32),
        pltpu.SemaphoreType.DMA,
    ],
)
def _mp_kernel(hs_hbm, src_hbm, dst_hbm, z64_hbm, out_hbm,
               sidx_v, didx_v, gbuf, hs_sh, acc_sh, sem):
    _mp_body(hs_hbm, src_hbm, dst_hbm, z64_hbm, out_hbm,
             sidx_v, didx_v, gbuf, hs_sh, acc_sh, sem)


def _tc_a_body(x_ref, w1_ref, dp_ref, hs_ref, dis_ref):
    deg = dp_ref[0, 0:N, 0:1] + dp_ref[1, 0:N, 0:1] + 1.0
    dis = lax.rsqrt(deg)
    dis_ref[...] = dis
    h = jnp.dot(x_ref[...], w1_ref[...], preferred_element_type=jnp.float32)
    hs_ref[0:N, :] = (dis * h).astype(jnp.float32)
    hs_ref[N:NP, :] = jnp.zeros((NP - N, DH), jnp.float32)


def _tc_mid_body(p_ref, hs_ref, dis_ref, b_ref, g_ref, be_ref, w2_ref, out_ref,
                 *, eps=1e-5):
    dis = dis_ref[...]
    acc = (p_ref[0, 0:N, :]
           + p_ref[1, 0:N, :].astype(jnp.float32)
           + hs_ref[0:N, :].astype(jnp.float32))
    z = dis * acc + b_ref[...]
    m = jnp.mean(z, axis=0, keepdims=True)
    v = jnp.mean((z - m) ** 2, axis=0, keepdims=True)
    zn = g_ref[...] * (z - m) * lax.rsqrt(v + eps) + be_ref[...]
    h = jnp.maximum(zn, 0.0)
    h2 = jnp.dot(h, w2_ref[...], preferred_element_type=jnp.float32)
    out_ref[0:N, :] = (dis * h2).astype(jnp.float32)
    out_ref[N:NP, :] = jnp.zeros((NP - N, DH), jnp.float32)


def _tc_out_body(p_ref, hs_ref, dis_ref, b_ref, g_ref, be_ref, out_ref,
                 *, eps=1e-5):
    dis = dis_ref[...]
    acc = (p_ref[0, 0:N, :]
           + p_ref[1, 0:N, :].astype(jnp.float32)
           + hs_ref[0:N, :].astype(jnp.float32))
    z = dis * acc + b_ref[...]
    m = jnp.mean(z, axis=0, keepdims=True)
    v = jnp.mean((z - m) ** 2, axis=0, keepdims=True)
    zn = g_ref[...] * (z - m) * lax.rsqrt(v + eps) + be_ref[...]
    out_ref[...] = jnp.maximum(zn, 0.0)


def kernel(x, edge_index, W1, b1, gamma1, beta1, W2, b2, gamma2, beta2):
    pad = jnp.full((E_PAD - E,), N, dtype=jnp.int32)
    src = jnp.concatenate([edge_index[0], pad]).reshape(NW, NCH * CHUNK)
    dst = jnp.concatenate([edge_index[1], pad]).reshape(NW, NCH * CHUNK)
    dst_deg = dst.reshape(NW, NCH, CHUNK)
    ones16 = jnp.ones((CHUNK, 16), jnp.float32)
    z16 = jnp.zeros((NP, 16), jnp.float32)
    z64 = jnp.zeros((NP, DH), jnp.float32)

    degpart = _deg_kernel(dst_deg, ones16, z16)

    hs1, dis = pl.pallas_call(
        _tc_a_body,
        out_shape=(jax.ShapeDtypeStruct((NP, DH), jnp.float32),
                   jax.ShapeDtypeStruct((N, 1), jnp.float32)),
    )(x, W1, degpart)

    p1 = _mp_kernel(hs1, src, dst, z64)

    hs2 = pl.pallas_call(
        _tc_mid_body,
        out_shape=jax.ShapeDtypeStruct((NP, DH), jnp.float32),
    )(p1, hs1, dis, b1, gamma1, beta1, W2)

    p2 = _mp_kernel(hs2, src, dst, z64)

    out = pl.pallas_call(
        _tc_out_body,
        out_shape=jax.ShapeDtypeStruct((N, DH), jnp.float32),
    )(p2, hs2, dis, b2, gamma2, beta2)

    return out


# speedup vs baseline: 1.7505x; 1.0205x over previous
import functools

import jax
import jax.numpy as jnp
from jax import lax
from jax.experimental import pallas as pl
from jax.experimental.pallas import tpu as pltpu
from jax.experimental.pallas import tpu_sc as plsc

N = 10000
E = 320000
D_IN = 128
DH = 64

NC = 2
NS = 16
NW = NC * NS
CHUNK = 128
NCH = 80
E_PAD = NW * NCH * CHUNK
NP = 10112
RPT = NP // NS
MB = 2

_MESH = plsc.VectorSubcoreMesh(core_axis_name="c", subcore_axis_name="s")


def _deg_body(dst_hbm, ones_hbm, z16_hbm, out_hbm, didx_v, ones_v, acc_sh, sem):
    c = lax.axis_index("c")
    s = lax.axis_index("s")
    wid = c * NS + s
    base = s * RPT
    pltpu.sync_copy(dst_hbm.at[wid], didx_v)
    pltpu.sync_copy(ones_hbm, ones_v)
    pltpu.sync_copy(z16_hbm.at[pl.ds(base, RPT)], acc_sh.at[pl.ds(base, RPT)])
    plsc.subcore_barrier()

    def body(j, carry):
        pltpu.sync_copy(ones_v, acc_sh.at[didx_v.at[j]], add=True)
        return carry

    lax.fori_loop(0, NCH, body, 0)
    plsc.subcore_barrier()
    pltpu.sync_copy(acc_sh.at[pl.ds(base, RPT)], out_hbm.at[c, pl.ds(base, RPT)])


@functools.partial(
    pl.kernel,
    out_type=jax.ShapeDtypeStruct((NC, NP, 16), jnp.float32),
    mesh=_MESH,
    compiler_params=pltpu.CompilerParams(use_tc_tiling_on_sc=False),
    scratch_types=[
        pltpu.VMEM((NCH, CHUNK), jnp.int32),
        pltpu.VMEM((CHUNK, 16), jnp.float32),
        pltpu.VMEM_SHARED((NP, 16), jnp.float32),
        pltpu.SemaphoreType.DMA,
    ],
)
def _deg_kernel(dst_hbm, ones_hbm, z16_hbm, out_hbm, didx_v, ones_v, acc_sh, sem):
    _deg_body(dst_hbm, ones_hbm, z16_hbm, out_hbm, didx_v, ones_v, acc_sh, sem)


def _mp_body(hs_hbm, src_hbm, dst_hbm, z64_hbm, out_hbm,
             sidx_v, didx_v, gbuf, hs_sh, acc_sh, sem):
    c = lax.axis_index("c")
    s = lax.axis_index("s")
    wid = c * NS + s
    base = s * RPT
    stages = [
        pltpu.async_copy(src_hbm.at[wid], sidx_v, sem),
        pltpu.async_copy(dst_hbm.at[wid], didx_v, sem),
        pltpu.async_copy(hs_hbm.at[pl.ds(base, RPT)],
                         hs_sh.at[pl.ds(base, RPT)], sem),
        pltpu.async_copy(z64_hbm.at[pl.ds(base, RPT)],
                         acc_sh.at[pl.ds(base, RPT)], sem),
    ]
    for d in stages:
        d.wait()
    plsc.subcore_barrier()

    def body(j, carry):
        blk = pl.ds(j * MB * CHUNK, MB * CHUNK)
        pltpu.async_copy(hs_sh.at[sidx_v.at[blk]], gbuf, sem).wait()
        pltpu.sync_copy(gbuf, acc_sh.at[didx_v.at[blk]], add=True)
        return carry

    lax.fori_loop(0, NCH // MB, body, 0)
    plsc.subcore_barrier()
    pltpu.sync_copy(acc_sh.at[pl.ds(base, RPT)], out_hbm.at[c, pl.ds(base, RPT)])


@functools.partial(
    pl.kernel,
    out_type=jax.ShapeDtypeStruct((NC, NP, DH), jnp.float32),
    mesh=_MESH,
    compiler_params=pltpu.CompilerParams(use_tc_tiling_on_sc=False),
    scratch_types=[
        pltpu.VMEM((NCH * CHUNK,), jnp.int32),
        pltpu.VMEM((NCH * CHUNK,), jnp.int32),
        pltpu.VMEM((MB * CHUNK, DH), jnp.float32),
        pltpu.VMEM_SHARED((NP, DH), jnp.float32),
        pltpu.VMEM_SHARED((NP, DH), jnp.float32),
        pltpu.SemaphoreType.DMA,
    ],
)
def _mp_kernel(hs_hbm, src_hbm, dst_hbm, z64_hbm, out_hbm,
               sidx_v, didx_v, gbuf, hs_sh, acc_sh, sem):
    _mp_body(hs_hbm, src_hbm, dst_hbm, z64_hbm, out_hbm,
             sidx_v, didx_v, gbuf, hs_sh, acc_sh, sem)


def _tc_a_body(x_ref, w1_ref, dp_ref, hs_ref, dis_ref):
    deg = dp_ref[0, 0:N, 0:1] + dp_ref[1, 0:N, 0:1] + 1.0
    dis = lax.rsqrt(deg)
    dis_ref[...] = dis
    h = jnp.dot(x_ref[...], w1_ref[...], preferred_element_type=jnp.float32)
    hs_ref[0:N, :] = (dis * h).astype(jnp.float32)
    hs_ref[N:NP, :] = jnp.zeros((NP - N, DH), jnp.float32)


def _tc_mid_body(p_ref, hs_ref, dis_ref, b_ref, g_ref, be_ref, w2_ref, out_ref,
                 *, eps=1e-5):
    dis = dis_ref[...]
    acc = (p_ref[0, 0:N, :]
           + p_ref[1, 0:N, :].astype(jnp.float32)
           + hs_ref[0:N, :].astype(jnp.float32))
    z = dis * acc + b_ref[...]
    m = jnp.mean(z, axis=0, keepdims=True)
    v = jnp.mean((z - m) ** 2, axis=0, keepdims=True)
    zn = g_ref[...] * (z - m) * lax.rsqrt(v + eps) + be_ref[...]
    h = jnp.maximum(zn, 0.0)
    h2 = jnp.dot(h, w2_ref[...], preferred_element_type=jnp.float32)
    out_ref[0:N, :] = (dis * h2).astype(jnp.float32)
    out_ref[N:NP, :] = jnp.zeros((NP - N, DH), jnp.float32)


def _tc_out_body(p_ref, hs_ref, dis_ref, b_ref, g_ref, be_ref, out_ref,
                 *, eps=1e-5):
    dis = dis_ref[...]
    acc = (p_ref[0, 0:N, :]
           + p_ref[1, 0:N, :].astype(jnp.float32)
           + hs_ref[0:N, :].astype(jnp.float32))
    z = dis * acc + b_ref[...]
    m = jnp.mean(z, axis=0, keepdims=True)
    v = jnp.mean((z - m) ** 2, axis=0, keepdims=True)
    zn = g_ref[...] * (z - m) * lax.rsqrt(v + eps) + be_ref[...]
    out_ref[...] = jnp.maximum(zn, 0.0)


def kernel(x, edge_index, W1, b1, gamma1, beta1, W2, b2, gamma2, beta2):
    pad = jnp.full((E_PAD - E,), N, dtype=jnp.int32)
    src = jnp.concatenate([edge_index[0], pad]).reshape(NW, NCH * CHUNK)
    dst = jnp.concatenate([edge_index[1], pad]).reshape(NW, NCH * CHUNK)
    dst_deg = dst.reshape(NW, NCH, CHUNK)
    ones16 = jnp.ones((CHUNK, 16), jnp.float32)
    z16 = jnp.zeros((NP, 16), jnp.float32)
    z64 = jnp.zeros((NP, DH), jnp.float32)

    degpart = _deg_kernel(dst_deg, ones16, z16)

    hs1, dis = pl.pallas_call(
        _tc_a_body,
        out_shape=(jax.ShapeDtypeStruct((NP, DH), jnp.float32),
                   jax.ShapeDtypeStruct((N, 1), jnp.float32)),
    )(x, W1, degpart)

    p1 = _mp_kernel(hs1, src, dst, z64)

    hs2 = pl.pallas_call(
        _tc_mid_body,
        out_shape=jax.ShapeDtypeStruct((NP, DH), jnp.float32),
    )(p1, hs1, dis, b1, gamma1, beta1, W2)

    p2 = _mp_kernel(hs2, src, dst, z64)

    out = pl.pallas_call(
        _tc_out_body,
        out_shape=jax.ShapeDtypeStruct((N, DH), jnp.float32),
    )(p2, hs2, dis, b2, gamma2, beta2)

    return out


# speedup vs baseline: 2.1064x; 1.2033x over previous
import functools

import jax
import jax.numpy as jnp
from jax import lax
from jax.experimental import pallas as pl
from jax.experimental.pallas import tpu as pltpu
from jax.experimental.pallas import tpu_sc as plsc

N = 10000
E = 320000
D_IN = 128
DH = 64

NC = 2
NS = 16
NW = NC * NS
CHUNK = 128
NCH = 80
E_PAD = NW * NCH * CHUNK
NP = 10112
RPT = NP // NS
MB = 1

_MESH = plsc.VectorSubcoreMesh(core_axis_name="c", subcore_axis_name="s")


def _deg_body(dst_hbm, ones_hbm, z16_hbm, out_hbm, didx_v, ones_v, acc_sh, sem):
    c = lax.axis_index("c")
    s = lax.axis_index("s")
    wid = c * NS + s
    base = s * RPT
    pltpu.sync_copy(dst_hbm.at[wid], didx_v)
    pltpu.sync_copy(ones_hbm, ones_v)
    pltpu.sync_copy(z16_hbm.at[pl.ds(base, RPT)], acc_sh.at[pl.ds(base, RPT)])
    plsc.subcore_barrier()

    def body(j, carry):
        pltpu.sync_copy(ones_v, acc_sh.at[didx_v.at[j]], add=True)
        return carry

    lax.fori_loop(0, NCH, body, 0)
    plsc.subcore_barrier()
    pltpu.sync_copy(acc_sh.at[pl.ds(base, RPT)], out_hbm.at[c, pl.ds(base, RPT)])


@functools.partial(
    pl.kernel,
    out_type=jax.ShapeDtypeStruct((NC, NP, 16), jnp.float32),
    mesh=_MESH,
    compiler_params=pltpu.CompilerParams(use_tc_tiling_on_sc=False),
    scratch_types=[
        pltpu.VMEM((NCH, CHUNK), jnp.int32),
        pltpu.VMEM((CHUNK, 16), jnp.float32),
        pltpu.VMEM_SHARED((NP, 16), jnp.float32),
        pltpu.SemaphoreType.DMA,
    ],
)
def _deg_kernel(dst_hbm, ones_hbm, z16_hbm, out_hbm, didx_v, ones_v, acc_sh, sem):
    _deg_body(dst_hbm, ones_hbm, z16_hbm, out_hbm, didx_v, ones_v, acc_sh, sem)


def _mp_body(hs_hbm, src_hbm, dst_hbm, z64_hbm, out_hbm,
             sidx_v, didx_v, gbuf, gbuf2, hs_sh, acc_sh, sem, sem2):
    c = lax.axis_index("c")
    s = lax.axis_index("s")
    wid = c * NS + s
    base = s * RPT
    stages = [
        pltpu.async_copy(src_hbm.at[wid], sidx_v, sem),
        pltpu.async_copy(dst_hbm.at[wid], didx_v, sem),
        pltpu.async_copy(hs_hbm.at[pl.ds(base, RPT)],
                         hs_sh.at[pl.ds(base, RPT)], sem),
        pltpu.async_copy(z64_hbm.at[pl.ds(base, RPT)],
                         acc_sh.at[pl.ds(base, RPT)], sem),
    ]
    for d in stages:
        d.wait()
    plsc.subcore_barrier()

    nmb = NCH // MB

    def blk(j):
        return pl.ds(j * MB * CHUNK, MB * CHUNK)

    pltpu.async_copy(hs_sh.at[sidx_v.at[blk(0)]], gbuf, sem)

    def body(i, carry):
        j0 = 2 * i
        pltpu.make_async_copy(hs_sh.at[sidx_v.at[blk(j0)]], gbuf, sem).wait()
        pltpu.async_copy(hs_sh.at[sidx_v.at[blk(j0 + 1)]], gbuf2, sem2)
        pltpu.sync_copy(gbuf, acc_sh.at[didx_v.at[blk(j0)]], add=True)
        jn = jnp.minimum(j0 + 2, nmb - 1)
        pltpu.make_async_copy(hs_sh.at[sidx_v.at[blk(j0 + 1)]], gbuf2, sem2).wait()
        pltpu.async_copy(hs_sh.at[sidx_v.at[blk(jn)]], gbuf, sem)
        pltpu.sync_copy(gbuf2, acc_sh.at[didx_v.at[blk(j0 + 1)]], add=True)
        return carry

    lax.fori_loop(0, nmb // 2, body, 0)
    pltpu.make_async_copy(hs_sh.at[sidx_v.at[blk(nmb - 1)]], gbuf, sem).wait()
    plsc.subcore_barrier()
    pltpu.sync_copy(acc_sh.at[pl.ds(base, RPT)], out_hbm.at[c, pl.ds(base, RPT)])


@functools.partial(
    pl.kernel,
    out_type=jax.ShapeDtypeStruct((NC, NP, DH), jnp.float32),
    mesh=_MESH,
    compiler_params=pltpu.CompilerParams(use_tc_tiling_on_sc=False),
    scratch_types=[
        pltpu.VMEM((NCH * CHUNK,), jnp.int32),
        pltpu.VMEM((NCH * CHUNK,), jnp.int32),
        pltpu.VMEM((MB * CHUNK, DH), jnp.float32),
        pltpu.VMEM((MB * CHUNK, DH), jnp.float32),
        pltpu.VMEM_SHARED((NP, DH), jnp.float32),
        pltpu.VMEM_SHARED((NP, DH), jnp.float32),
        pltpu.SemaphoreType.DMA,
        pltpu.SemaphoreType.DMA,
    ],
)
def _mp_kernel(hs_hbm, src_hbm, dst_hbm, z64_hbm, out_hbm,
               sidx_v, didx_v, gbuf, gbuf2, hs_sh, acc_sh, sem, sem2):
    _mp_body(hs_hbm, src_hbm, dst_hbm, z64_hbm, out_hbm,
             sidx_v, didx_v, gbuf, gbuf2, hs_sh, acc_sh, sem, sem2)


def _tc_a_body(x_ref, w1_ref, dp_ref, hs_ref, dis_ref):
    deg = dp_ref[0, 0:N, 0:1] + dp_ref[1, 0:N, 0:1] + 1.0
    dis = lax.rsqrt(deg)
    dis_ref[...] = dis
    h = jnp.dot(x_ref[...], w1_ref[...], preferred_element_type=jnp.float32)
    hs_ref[0:N, :] = (dis * h).astype(jnp.float32)
    hs_ref[N:NP, :] = jnp.zeros((NP - N, DH), jnp.float32)


def _tc_mid_body(p_ref, hs_ref, dis_ref, b_ref, g_ref, be_ref, w2_ref, out_ref,
                 *, eps=1e-5):
    dis = dis_ref[...]
    acc = (p_ref[0, 0:N, :]
           + p_ref[1, 0:N, :].astype(jnp.float32)
           + hs_ref[0:N, :].astype(jnp.float32))
    z = dis * acc + b_ref[...]
    m = jnp.mean(z, axis=0, keepdims=True)
    v = jnp.mean((z - m) ** 2, axis=0, keepdims=True)
    zn = g_ref[...] * (z - m) * lax.rsqrt(v + eps) + be_ref[...]
    h = jnp.maximum(zn, 0.0)
    h2 = jnp.dot(h, w2_ref[...], preferred_element_type=jnp.float32)
    out_ref[0:N, :] = (dis * h2).astype(jnp.float32)
    out_ref[N:NP, :] = jnp.zeros((NP - N, DH), jnp.float32)


def _tc_out_body(p_ref, hs_ref, dis_ref, b_ref, g_ref, be_ref, out_ref,
                 *, eps=1e-5):
    dis = dis_ref[...]
    acc = (p_ref[0, 0:N, :]
           + p_ref[1, 0:N, :].astype(jnp.float32)
           + hs_ref[0:N, :].astype(jnp.float32))
    z = dis * acc + b_ref[...]
    m = jnp.mean(z, axis=0, keepdims=True)
    v = jnp.mean((z - m) ** 2, axis=0, keepdims=True)
    zn = g_ref[...] * (z - m) * lax.rsqrt(v + eps) + be_ref[...]
    out_ref[...] = jnp.maximum(zn, 0.0)


def kernel(x, edge_index, W1, b1, gamma1, beta1, W2, b2, gamma2, beta2):
    pad = jnp.full((E_PAD - E,), N, dtype=jnp.int32)
    src = jnp.concatenate([edge_index[0], pad]).reshape(NW, NCH * CHUNK)
    dst = jnp.concatenate([edge_index[1], pad]).reshape(NW, NCH * CHUNK)
    dst_deg = dst.reshape(NW, NCH, CHUNK)
    ones16 = jnp.ones((CHUNK, 16), jnp.float32)
    z16 = jnp.zeros((NP, 16), jnp.float32)
    z64 = jnp.zeros((NP, DH), jnp.float32)

    degpart = _deg_kernel(dst_deg, ones16, z16)

    hs1, dis = pl.pallas_call(
        _tc_a_body,
        out_shape=(jax.ShapeDtypeStruct((NP, DH), jnp.float32),
                   jax.ShapeDtypeStruct((N, 1), jnp.float32)),
    )(x, W1, degpart)

    p1 = _mp_kernel(hs1, src, dst, z64)

    hs2 = pl.pallas_call(
        _tc_mid_body,
        out_shape=jax.ShapeDtypeStruct((NP, DH), jnp.float32),
    )(p1, hs1, dis, b1, gamma1, beta1, W2)

    p2 = _mp_kernel(hs2, src, dst, z64)

    out = pl.pallas_call(
        _tc_out_body,
        out_shape=jax.ShapeDtypeStruct((N, DH), jnp.float32),
    )(p2, hs2, dis, b2, gamma2, beta2)

    return out
